# baseline (speedup 1.0000x reference)
import functools

import jax
import jax.numpy as jnp
from jax import lax
from jax.experimental import pallas as pl
from jax.experimental.pallas import tpu as pltpu
from jax.experimental.pallas import tpu_sc as plsc

VOCAB = 100000
D = 64
B = 4096
L = 200

NC = 2
NS = 16
NW = NC * NS

ROWS = B * L
RPW = ROWS // NW
CHUNK = 100
NCHUNK = RPW // CHUNK


NPAIR = NCHUNK // 2
NBUF = 2


def _sc_kernel(fused_hbm, idx_hbm, pe_hbm, out_hbm, idx_v, pe_v, gbuf, obuf,
               g0, g1, o0, o1):
    wid = lax.axis_index("s") * NC + lax.axis_index("c")
    gsem = (g0, g1)
    osem = (o0, o1)

    IDXC = 32 * L

    def i_load(q):
        pltpu.sync_copy(idx_hbm.at[wid, pl.ds(q * IDXC, IDXC)],
                        idx_v.at[pl.ds((q % 2) * IDXC, IDXC)])

    i_load(0)
    i_load(1)
    pltpu.sync_copy(pe_hbm, pe_v)

    SPLIT = 104

    def g_copy(b, c, t):
        lo = 0 if t == 0 else SPLIT
        n = SPLIT if t == 0 else L - SPLIT
        return pltpu.make_async_copy(
            fused_hbm.at[idx_v.at[pl.ds(
                ((c // 32) % 2) * IDXC + (c % 32) * L + lo, n)]],
            gbuf.at[b, pl.ds(lo, n)],
            gsem[b])

    def g_start(b, c):
        for t in range(2):
            g_copy(b, c, t).start()

    def g_wait(b, c):
        for t in range(2):
            g_copy(b, c, t).wait()

    def o_copy(b, c):
        return pltpu.make_async_copy(
            obuf.at[b], out_hbm.at[wid * NPAIR + c], osem[b])

    def add_pe(b):
        @plsc.parallel_loop(0, L, unroll=2)
        def _(l):
            for g in range(4):
                sl = pl.ds(g * 16, 16)
                obuf[b, l, sl] = (
                    gbuf[b, l, sl] + pe_v[pl.ds(l * D + g * 16, 16)])

    for b in range(NBUF):
        g_start(b, b)

    for b in range(NBUF):
        g_wait(b, b)
        add_pe(b)
        o_copy(b, b).start()
        g_start(b, b + NBUF)

    def steady(o, carry):
        for b in range(NBUF):
            c = o * NBUF + b
            if b == 0:
                @pl.when((c % 32 == 0) & (c >= 32) & (c < 96))
                def _():
                    i_load(c // 32 + 1)
            g_wait(b, c)
            o_copy(b, c - NBUF).wait()
            add_pe(b)
            o_copy(b, c).start()
            g_start(b, c + NBUF)
        return carry

    lax.fori_loop(1, NPAIR // NBUF - 1, steady, 0)

    for b in range(NBUF):
        c = NPAIR - NBUF + b
        g_wait(b, c)
        o_copy(b, c - NBUF).wait()
        add_pe(b)
        o_copy(b, c).start()
    for b in range(NBUF):
        o_copy(b, NPAIR - NBUF + b).wait()


@jax.jit
def kernel(x, special_table, molecule_table, pe):
    s0 = special_table[0:1]
    fused = jnp.pad(molecule_table, ((0, 0), (0, D))).at[:4, :D].add(
        special_table - s0)
    pe_eff = (pe[0] + s0).reshape(-1)
    idx = x.astype(jnp.int32).reshape(NW, RPW)

    mesh = plsc.VectorSubcoreMesh(core_axis_name="c", subcore_axis_name="s")
    out = pl.kernel(
        _sc_kernel,
        out_type=jax.ShapeDtypeStruct((B, L, D), jnp.float32),
        mesh=mesh,
        compiler_params=pltpu.CompilerParams(use_tc_tiling_on_sc=True),
        scratch_types=[
            pltpu.VMEM((2 * 32 * L,), jnp.int32),
            pltpu.VMEM((L * D,), jnp.float32),
            pltpu.VMEM((NBUF, L, 2 * D), jnp.float32),
            pltpu.VMEM((NBUF, L, D), jnp.float32),
            pltpu.SemaphoreType.DMA,
            pltpu.SemaphoreType.DMA,
            pltpu.SemaphoreType.DMA,
            pltpu.SemaphoreType.DMA,
        ],
    )(fused, idx, pe_eff)
    return out


# speedup vs baseline: 1.5452x; 1.5452x over previous
import functools

import jax
import jax.numpy as jnp
from jax import lax
from jax.experimental import pallas as pl
from jax.experimental.pallas import tpu as pltpu
from jax.experimental.pallas import tpu_sc as plsc

VOCAB = 100000
D = 64
B = 4096
L = 200

NC = 2
NS = 16
NW = NC * NS

ROWS = B * L
RPW = ROWS // NW
CHUNK = 100
NCHUNK = RPW // CHUNK


NPAIR = NCHUNK // 2
NBUF = 2


def _sc_kernel(fused_hbm, idx_hbm, pe_hbm, out_hbm, idx_v, pe_v, gbuf, obuf,
               g0, g1, o0, o1):
    wid = lax.axis_index("s") * NC + lax.axis_index("c")
    gsem = (g0, g1)
    osem = (o0, o1)

    pltpu.sync_copy(idx_hbm.at[pl.ds(wid * NPAIR, NPAIR)], idx_v)
    pltpu.sync_copy(pe_hbm, pe_v)

    SPLIT = 104

    def g_copy(b, c, t):
        lo = 0 if t == 0 else SPLIT
        n = SPLIT if t == 0 else L - SPLIT
        return pltpu.make_async_copy(
            fused_hbm.at[idx_v.at[c, pl.ds(lo, n)]],
            gbuf.at[b, pl.ds(lo, n)], gsem[b])

    def g_start(b, c):
        for t in range(2):
            g_copy(b, c, t).start()

    def g_wait(b, c):
        for t in range(2):
            g_copy(b, c, t).wait()

    def o_copy(b, c):
        return pltpu.make_async_copy(
            obuf.at[b],
            out_hbm.at[wid * NPAIR + c, :, pl.ds(0, D)],
            osem[b])

    def add_pe(b):
        @plsc.parallel_loop(0, L, unroll=4)
        def _(j):
            for g in range(4):
                sl = pl.ds(g * 16, 16)
                obuf[b, j, sl] = gbuf[b, j, sl] + pe_v[j, sl]

    for b in range(NBUF):
        g_start(b, b)

    for b in range(NBUF):
        g_wait(b, b)
        add_pe(b)
        o_copy(b, b).start()
        g_start(b, b + NBUF)

    def steady(o, carry):
        for b in range(NBUF):
            c = o * NBUF + b
            g_wait(b, c)
            o_copy(b, c - NBUF).wait()
            add_pe(b)
            o_copy(b, c).start()
            g_start(b, c + NBUF)
        return carry

    lax.fori_loop(1, NPAIR // NBUF - 1, steady, 0)

    for b in range(NBUF):
        c = NPAIR - NBUF + b
        g_wait(b, c)
        o_copy(b, c - NBUF).wait()
        add_pe(b)
        o_copy(b, c).start()
    for b in range(NBUF):
        o_copy(b, NPAIR - NBUF + b).wait()


@jax.jit
def kernel(x, special_table, molecule_table, pe):
    s0 = special_table[0:1]
    fused = molecule_table.at[:4].add(special_table - s0)
    pe_eff = pe[0] + s0
    idx = x.astype(jnp.int32)

    mesh = plsc.VectorSubcoreMesh(core_axis_name="c", subcore_axis_name="s")
    out = pl.kernel(
        _sc_kernel,
        out_type=jax.ShapeDtypeStruct((B, L, 2 * D), jnp.float32),
        mesh=mesh,
        compiler_params=pltpu.CompilerParams(use_tc_tiling_on_sc=False),
        scratch_types=[
            pltpu.VMEM((NPAIR, L), jnp.int32),
            pltpu.VMEM((L, D), jnp.float32),
            pltpu.VMEM((NBUF, L, D), jnp.float32),
            pltpu.VMEM((NBUF, L, D), jnp.float32),
            pltpu.SemaphoreType.DMA,
            pltpu.SemaphoreType.DMA,
            pltpu.SemaphoreType.DMA,
            pltpu.SemaphoreType.DMA,
        ],
    )(fused, idx, pe_eff)
    return out[:, :, :D]


# speedup vs baseline: 1.5476x; 1.0015x over previous
import jax
import jax.numpy as jnp
from jax import lax
from jax.experimental import pallas as pl
from jax.experimental.pallas import tpu as pltpu
from jax.experimental.pallas import tpu_sc as plsc

VOCAB = 100000
D = 64
B = 4096
L = 200

NC = 2
NS = 16
NW = NC * NS

ROWS = B * L
RPW = ROWS // NW
CHUNK = 100
NCHUNK = RPW // CHUNK

NPAIR = NCHUNK // 2
NBUF = 2


def _sc_kernel(fused_hbm, idx_hbm, pe_hbm, out_hbm, idx_v, pe_v, gbuf, obuf,
               g0, g1, o0, o1):
    wid = lax.axis_index("s") * NC + lax.axis_index("c")
    gsem = (g0, g1)
    osem = (o0, o1)

    pltpu.sync_copy(idx_hbm.at[pl.ds(wid * NPAIR, NPAIR)], idx_v)
    pltpu.sync_copy(pe_hbm, pe_v)

    SPLIT = 104

    def g_copy(b, c, t):
        lo = 0 if t == 0 else SPLIT
        n = SPLIT if t == 0 else L - SPLIT
        return pltpu.make_async_copy(
            fused_hbm.at[idx_v.at[c, pl.ds(lo, n)]],
            gbuf.at[b, pl.ds(lo, n)], gsem[b])

    def g_start(b, c):
        for t in range(2):
            g_copy(b, c, t).start()

    def g_wait(b, c):
        for t in range(2):
            g_copy(b, c, t).wait()

    def o_copy(b, c):
        return pltpu.make_async_copy(
            obuf.at[b],
            out_hbm.at[wid * NPAIR + c, :, pl.ds(0, D)],
            osem[b])

    def add_pe(b):
        @plsc.parallel_loop(0, L, unroll=4)
        def _(j):
            for g in range(4):
                sl = pl.ds(g * 16, 16)
                obuf[b, j, sl] = gbuf[b, j, sl] + pe_v[j, sl]

    for b in range(NBUF):
        g_start(b, b)

    for b in range(NBUF):
        g_wait(b, b)
        add_pe(b)
        o_copy(b, b).start()
        g_start(b, b + NBUF)

    def steady(o, carry):
        for b in range(NBUF):
            c = o * NBUF + b
            g_wait(b, c)
            o_copy(b, c - NBUF).wait()
            add_pe(b)
            o_copy(b, c).start()
            g_start(b, c + NBUF)
        return carry

    lax.fori_loop(1, NPAIR // NBUF - 1, steady, 0)

    for b in range(NBUF):
        c = NPAIR - NBUF + b
        g_wait(b, c)
        o_copy(b, c - NBUF).wait()
        add_pe(b)
        o_copy(b, c).start()
    for b in range(NBUF):
        o_copy(b, NPAIR - NBUF + b).wait()


@jax.jit
def kernel(x, special_table, molecule_table, pe):
    s0 = special_table[0:1]
    fused = molecule_table.at[:4].add(special_table - s0)
    pe_eff = pe[0] + s0
    idx = x.astype(jnp.int32)

    mesh = plsc.VectorSubcoreMesh(core_axis_name="c", subcore_axis_name="s")
    out = pl.kernel(
        _sc_kernel,
        out_type=jax.ShapeDtypeStruct((B, L, 2 * D), jnp.float32),
        mesh=mesh,
        compiler_params=pltpu.CompilerParams(use_tc_tiling_on_sc=False),
        scratch_types=[
            pltpu.VMEM((NPAIR, L), jnp.int32),
            pltpu.VMEM((L, D), jnp.float32),
            pltpu.VMEM((NBUF, L, D), jnp.float32),
            pltpu.VMEM((NBUF, L, D), jnp.float32),
            pltpu.SemaphoreType.DMA,
            pltpu.SemaphoreType.DMA,
            pltpu.SemaphoreType.DMA,
            pltpu.SemaphoreType.DMA,
        ],
    )(fused, idx, pe_eff)
    return out[:, :, :D]
